# baseline (speedup 1.0000x reference)
import functools

import jax
import jax.numpy as jnp
from jax import lax
from jax.experimental import pallas as pl
from jax.experimental.pallas import tpu as pltpu
from jax.experimental.pallas import tpu_sc as plsc

NUM_CORES = 2
NUM_SUBCORES = 16
NUM_WORKERS = NUM_CORES * NUM_SUBCORES
CHUNK = 128
DIM = 64
NBUF = 8


def _make_kernel(n_rows: int, n_tok: int):
    mesh = plsc.VectorSubcoreMesh(core_axis_name="c", subcore_axis_name="s")
    iblk = n_tok // NUM_WORKERS
    n_ml = iblk // CHUNK
    n_chunks = n_rows * n_ml

    @functools.partial(
        pl.kernel,
        out_type=jax.ShapeDtypeStruct((n_rows, n_tok, DIM), jnp.float32),
        mesh=mesh,
        scratch_types=[
            pltpu.VMEM((n_ml, n_rows, CHUNK), jnp.int32),
            pltpu.VMEM((NBUF, CHUNK, DIM), jnp.float32),
            pltpu.SemaphoreType.DMA,
            pltpu.SemaphoreType.DMA,
        ],
        compiler_params=pltpu.CompilerParams(use_tc_tiling_on_sc=False),
    )
    def k(x_hbm, w_hbm, out_hbm, idx_v, bufs, gsem, wsem):
        wid = lax.axis_index("s") * NUM_CORES + lax.axis_index("c")
        i0 = wid * iblk
        for ml in range(n_ml):
            pltpu.sync_copy(x_hbm.at[:, pl.ds(i0 + ml * CHUNK, CHUNK)],
                            idx_v.at[ml])

        def fire(n):
            j = n // n_ml
            ml = n % n_ml
            pltpu.async_copy(w_hbm.at[idx_v.at[ml, j]], bufs.at[n % NBUF],
                             gsem)

        for n in range(NBUF - 1):
            fire(n)

        @pl.loop(0, n_chunks)
        def _(n):
            j = n // n_ml
            ml = n % n_ml
            s = n % NBUF
            pltpu.make_async_copy(w_hbm.at[pl.ds(0, CHUNK)], bufs.at[s],
                                  gsem).wait()
            pltpu.async_copy(bufs.at[s],
                             out_hbm.at[j, pl.ds(i0 + ml * CHUNK, CHUNK)],
                             wsem)

            @pl.when(n + NBUF - 1 < n_chunks)
            def _():
                @pl.when(n >= 1)
                def _():
                    pltpu.make_async_copy(bufs.at[0],
                                          out_hbm.at[0, pl.ds(0, CHUNK)],
                                          wsem).wait()

                fire(n + NBUF - 1)

        for _ in range(NBUF):
            pltpu.make_async_copy(bufs.at[0], out_hbm.at[0, pl.ds(0, CHUNK)],
                                  wsem).wait()

    return k


def kernel(x, weight):
    rows, cols = x.shape
    xt = x.T.astype(jnp.int32)
    out = _make_kernel(cols, rows)(xt, weight)
    return out.transpose(1, 0, 2)
